# baseline (speedup 1.0000x reference)
import jax
import jax.numpy as jnp
from jax.experimental import pallas as pl
from jax.experimental.pallas import tpu as pltpu

_CHUNK = 256
_TN = 256
_MC = 2048


def _dot_t(a, b):
    return jax.lax.dot_general(
        a, b, dimension_numbers=(((1,), (1,)), ((), ())),
        preferred_element_type=jnp.float32)


def _mm_kernel(x_hbm, w_ref, o_ref, xb_ref, stage_ref, sems):
    j = pl.program_id(0)
    n_chunk = x_hbm.shape[0] // _CHUNK
    wb = w_ref[...].astype(jnp.bfloat16)

    @pl.when(j == 0)
    def _load_x_and_compute():
        def _copy(c, slot):
            return pltpu.make_async_copy(
                x_hbm.at[pl.ds(c * _CHUNK, _CHUNK), :],
                stage_ref.at[slot],
                sems.at[slot],
            )

        _copy(0, 0).start()

        def _body(c, carry):
            slot = jax.lax.rem(c, 2)

            @pl.when(c + 1 < n_chunk)
            def _():
                _copy(c + 1, 1 - slot).start()

            _copy(c, slot).wait()
            xc = stage_ref[slot].astype(jnp.bfloat16)
            xb_ref[pl.ds(c * _CHUNK, _CHUNK), :] = xc
            o_ref[pl.ds(c * _CHUNK, _CHUNK), :] = _dot_t(xc, wb)
            return carry

        jax.lax.fori_loop(0, n_chunk, _body, 0)

    @pl.when(j > 0)
    def _compute():
        for mc in range(x_hbm.shape[0] // _MC):
            o_ref[mc * _MC:(mc + 1) * _MC, :] = _dot_t(
                xb_ref[mc * _MC:(mc + 1) * _MC, :], wb)


@jax.jit
def kernel(x, weight):
    *lead, K = x.shape
    N = weight.shape[0]
    x2d = x.reshape(-1, K)
    M = x2d.shape[0]

    out2d = pl.pallas_call(
        _mm_kernel,
        out_shape=jax.ShapeDtypeStruct((M, N), x.dtype),
        grid=(N // _TN,),
        in_specs=[
            pl.BlockSpec(memory_space=pl.ANY),
            pl.BlockSpec((_TN, K), lambda j: (j, 0)),
        ],
        out_specs=pl.BlockSpec((M, _TN), lambda j: (0, j)),
        scratch_shapes=[
            pltpu.VMEM((M, K), jnp.bfloat16),
            pltpu.VMEM((2, _CHUNK, K), jnp.float32),
            pltpu.SemaphoreType.DMA((2,)),
        ],
        compiler_params=pltpu.CompilerParams(
            dimension_semantics=("arbitrary",),
            vmem_limit_bytes=62 << 20,
        ),
    )(x2d, weight)
    return out2d.reshape(*lead, N)


# speedup vs baseline: 1.0112x; 1.0112x over previous
import jax
import jax.numpy as jnp
from jax.experimental import pallas as pl
from jax.experimental.pallas import tpu as pltpu

_CHUNK = 256
_TN = 256
_MC = 1024


def _dot_t(a, b):
    return jax.lax.dot_general(
        a, b, dimension_numbers=(((1,), (1,)), ((), ())),
        preferred_element_type=jnp.float32)


def _mm_kernel(x_hbm, w_ref, o_ref, xb_ref, stage_ref, sems):
    j = pl.program_id(0)
    n_chunk = x_hbm.shape[0] // _CHUNK
    wb = w_ref[...].astype(jnp.bfloat16)

    @pl.when(j == 0)
    def _load_x_and_compute():
        def _copy(c, slot):
            return pltpu.make_async_copy(
                x_hbm.at[pl.ds(c * _CHUNK, _CHUNK), :],
                stage_ref.at[slot],
                sems.at[slot],
            )

        _copy(0, 0).start()

        def _body(c, carry):
            slot = jax.lax.rem(c, 2)

            @pl.when(c + 1 < n_chunk)
            def _():
                _copy(c + 1, 1 - slot).start()

            _copy(c, slot).wait()
            xc = stage_ref[slot].astype(jnp.bfloat16)
            xb_ref[pl.ds(c * _CHUNK, _CHUNK), :] = xc
            o_ref[pl.ds(c * _CHUNK, _CHUNK), :] = _dot_t(xc, wb)
            return carry

        jax.lax.fori_loop(0, n_chunk, _body, 0)

    @pl.when(j > 0)
    def _compute():
        for mc in range(x_hbm.shape[0] // _MC):
            o_ref[mc * _MC:(mc + 1) * _MC, :] = _dot_t(
                xb_ref[mc * _MC:(mc + 1) * _MC, :], wb)


@jax.jit
def kernel(x, weight):
    *lead, K = x.shape
    N = weight.shape[0]
    x2d = x.reshape(-1, K)
    M = x2d.shape[0]

    out2d = pl.pallas_call(
        _mm_kernel,
        out_shape=jax.ShapeDtypeStruct((M, N), x.dtype),
        grid=(N // _TN,),
        in_specs=[
            pl.BlockSpec(memory_space=pl.ANY),
            pl.BlockSpec((_TN, K), lambda j: (j, 0)),
        ],
        out_specs=pl.BlockSpec((M, _TN), lambda j: (0, j)),
        scratch_shapes=[
            pltpu.VMEM((M, K), jnp.bfloat16),
            pltpu.VMEM((2, _CHUNK, K), jnp.float32),
            pltpu.SemaphoreType.DMA((2,)),
        ],
        compiler_params=pltpu.CompilerParams(
            dimension_semantics=("arbitrary",),
            vmem_limit_bytes=61 << 20,
        ),
    )(x2d, weight)
    return out2d.reshape(*lead, N)
